# baseline (speedup 1.0000x reference)
import functools

import jax
import jax.numpy as jnp
from jax import lax
from jax.experimental import pallas as pl
from jax.experimental.pallas import tpu as pltpu
from jax.experimental.pallas import tpu_sc as plsc

N = 10000
D = 128
NC = 2
NS = 16
NW = NC * NS
CH = 128
CPT = 80
EP = NW * CPT * CH
RPT = 640
NPAD = NS * RPT
ZR = 64
NBUF = 2
GC = 8
NG = CPT // GC
CPT0 = 160
NG0 = CPT0 // GC

_MESH = dict(core_axis_name="c", subcore_axis_name="s")


def _deg_kernel(dst2d):
    @functools.partial(
        pl.kernel,
        out_type=jax.ShapeDtypeStruct((NC, NPAD), jnp.float32),
        mesh=plsc.VectorSubcoreMesh(**_MESH),
        scratch_types=[
            pltpu.VMEM((CPT, CH), jnp.int32),
            pltpu.VMEM((CH,), jnp.float32),
            pltpu.VMEM((RPT,), jnp.float32),
            pltpu.VMEM_SHARED((NPAD,), jnp.float32),
        ],
    )
    def k(dst_hbm, deg_hbm, dstv, ones_v, zb, acc):
        cid = lax.axis_index("c")
        sid = lax.axis_index("s")
        wid = sid * NC + cid
        for j in range(CH // 16):
            ones_v[pl.ds(16 * j, 16)] = jnp.ones((16,), jnp.float32)

        def zfill(i, carry):
            zb[pl.ds(i * 16, 16)] = jnp.zeros((16,), jnp.float32)
            return carry

        lax.fori_loop(0, RPT // 16, zfill, 0)
        pltpu.sync_copy(zb, acc.at[pl.ds(sid * RPT, RPT)])
        pltpu.sync_copy(dst_hbm.at[pl.ds(wid * CPT, CPT)], dstv)
        plsc.subcore_barrier()

        def body(j, carry):
            pltpu.sync_copy(ones_v, acc.at[dstv.at[j]], add=True)
            return carry

        lax.fori_loop(0, CPT, body, 0)
        plsc.subcore_barrier()
        pltpu.sync_copy(acc.at[pl.ds(sid * RPT, RPT)],
                        deg_hbm.at[cid].at[pl.ds(sid * RPT, RPT)])

    return k(dst2d)


def _scatter_kernel(y, src2d, dst2d):
    @functools.partial(
        pl.kernel,
        out_type=jax.ShapeDtypeStruct((NPAD, D), jnp.float32),
        mesh=plsc.VectorSubcoreMesh(**_MESH),
        scratch_types=[
            pltpu.VMEM((2 * GC, CH), jnp.int32),
            pltpu.VMEM((2 * GC, CH), jnp.int32),
            pltpu.VMEM((NBUF, CH, D), jnp.float32),
            pltpu.VMEM((ZR, D), jnp.float32),
            pltpu.VMEM_SHARED((NPAD, D), jnp.float32),
            [pltpu.SemaphoreType.DMA] * (NBUF + 4),
        ],
    )
    def k(y_hbm, src_hbm, dst_hbm, agg_hbm, sidx, didx, rows, zb, acc, sems):
        cid = lax.axis_index("c")
        sid = lax.axis_index("s")
        gsem = sems[:NBUF]
        isem = sems[NBUF:]
        base = sid * CPT0

        def zfill(i, carry):
            for j in range(D // 16):
                zb[i, pl.ds(16 * j, 16)] = jnp.zeros((16,), jnp.float32)
            return carry

        def idx_fetch(g, s):
            pltpu.async_copy(src_hbm.at[pl.ds(base + g * GC, GC)],
                             sidx.at[pl.ds(s * GC, GC)], isem[2 * s])
            pltpu.async_copy(dst_hbm.at[pl.ds(base + g * GC, GC)],
                             didx.at[pl.ds(s * GC, GC)], isem[2 * s + 1])

        def idx_wait(s):
            pltpu.make_async_copy(src_hbm.at[pl.ds(base, GC)],
                                  sidx.at[pl.ds(s * GC, GC)],
                                  isem[2 * s]).wait()
            pltpu.make_async_copy(dst_hbm.at[pl.ds(base, GC)],
                                  didx.at[pl.ds(s * GC, GC)],
                                  isem[2 * s + 1]).wait()

        def group(s, prefetch_g):
            idx_wait(s)
            cp = pltpu.async_copy(y_hbm.at[sidx.at[s * GC]], rows.at[0],
                                  gsem[0])
            for c in range(GC):
                nxt = None
                if c + 1 < GC:
                    nxt = pltpu.async_copy(y_hbm.at[sidx.at[s * GC + c + 1]],
                                           rows.at[(c + 1) % NBUF],
                                           gsem[(c + 1) % NBUF])
                cp.wait()
                pltpu.sync_copy(rows.at[c % NBUF],
                                acc.at[didx.at[s * GC + c]], add=True)
                cp = nxt
            idx_fetch(prefetch_g, s)

        @pl.when(cid == 0)
        def _():
            lax.fori_loop(0, ZR, zfill, 0)
            for c in range(RPT // ZR):
                pltpu.sync_copy(zb, acc.at[pl.ds(sid * RPT + c * ZR, ZR)])
            idx_fetch(0, 0)
            idx_fetch(1, 1)
            plsc.subcore_barrier()

            def body(g2, carry):
                g = g2 * 2
                group(0, jnp.minimum(g + 2, NG0 - 1))
                group(1, jnp.minimum(g + 3, NG0 - 1))
                return carry

            lax.fori_loop(0, NG0 // 2, body, 0)
            idx_wait(0)
            idx_wait(1)
            plsc.subcore_barrier()
            pltpu.sync_copy(acc.at[pl.ds(sid * RPT, RPT)],
                            agg_hbm.at[pl.ds(sid * RPT, RPT)])

    return k(y, src2d, dst2d)


def _tc0(x, W1):
    def body(x_ref, w_ref, xw_ref):
        xw_ref[...] = jnp.dot(x_ref[...], w_ref[...],
                              preferred_element_type=jnp.float32)

    return pl.pallas_call(
        body,
        out_shape=jax.ShapeDtypeStruct((N, D), jnp.float32),
    )(x, W1)


def _tc1(xw, degp):
    def body(xw_ref, degp_ref, y_ref, dis_ref, inv_ref):
        deg = degp_ref[0, :N, :] + degp_ref[1, :N, :] + 1.0
        dis = lax.rsqrt(deg)
        inv = 1.0 / deg
        y_ref[...] = xw_ref[...] * dis
        dis_ref[...] = dis
        inv_ref[...] = inv

    return pl.pallas_call(
        body,
        out_shape=[
            jax.ShapeDtypeStruct((N, D), jnp.float32),
            jax.ShapeDtypeStruct((N, 1), jnp.float32),
            jax.ShapeDtypeStruct((N, 1), jnp.float32),
        ],
    )(xw, degp)


def _agg_from_partials(aggp_ref):
    return aggp_ref[:N, :]


def _tc2(aggp, xw1, dis, inv, x, b1, Wg1, bg1, Wlin, blin, gamma1, beta1):
    def body(aggp_ref, xw1_ref, dis_ref, inv_ref, x_ref, b1_ref, Wg1_ref,
             bg1_ref, Wlin_ref, blin_ref, g1_ref, be1_ref, h_ref):
        agg = _agg_from_partials(aggp_ref)
        conv = agg * dis_ref[...] + xw1_ref[...] * inv_ref[...] + b1_ref[...]
        z = jnp.tanh(conv)
        g = jax.nn.sigmoid(
            jnp.dot(z, Wg1_ref[...], preferred_element_type=jnp.float32)
            + bg1_ref[...])
        h0 = jnp.dot(x_ref[...], Wlin_ref[...],
                     preferred_element_type=jnp.float32) + blin_ref[...]
        h = (1.0 - g) * h0 + g * z
        h = jnp.maximum(h, 0.0)
        m = jnp.mean(h, axis=0, keepdims=True)
        v = jnp.mean((h - m) * (h - m), axis=0, keepdims=True)
        h_ref[...] = (h - m) * lax.rsqrt(v + 1e-5) * g1_ref[...] + be1_ref[...]

    return pl.pallas_call(
        body,
        out_shape=jax.ShapeDtypeStruct((N, D), jnp.float32),
    )(aggp, xw1, dis, inv, x, b1, Wg1, bg1, Wlin, blin, gamma1, beta1)


def _tc2b(h, W2, dis):
    def body(h_ref, W2_ref, dis_ref, xw2_ref, y2_ref):
        xw2 = jnp.dot(h_ref[...], W2_ref[...],
                      preferred_element_type=jnp.float32)
        xw2_ref[...] = xw2
        y2_ref[...] = xw2 * dis_ref[...]

    return pl.pallas_call(
        body,
        out_shape=[
            jax.ShapeDtypeStruct((N, D), jnp.float32),
            jax.ShapeDtypeStruct((N, D), jnp.float32),
        ],
    )(h, W2, dis)


def _tc3(aggp, xw2, dis, inv, h1, b2, Wg2, bg2, gamma2, beta2):
    def body(aggp_ref, xw2_ref, dis_ref, inv_ref, h1_ref, b2_ref, Wg2_ref,
             bg2_ref, g2_ref, be2_ref, out_ref):
        agg = _agg_from_partials(aggp_ref)
        conv = agg * dis_ref[...] + xw2_ref[...] * inv_ref[...] + b2_ref[...]
        z = jnp.tanh(conv)
        g = jax.nn.sigmoid(
            jnp.dot(z, Wg2_ref[...], preferred_element_type=jnp.float32)
            + bg2_ref[...])
        h = (1.0 - g) * h1_ref[...] + g * z
        h = jnp.maximum(h, 0.0)
        m = jnp.mean(h, axis=0, keepdims=True)
        v = jnp.mean((h - m) * (h - m), axis=0, keepdims=True)
        out_ref[...] = (h - m) * lax.rsqrt(v + 1e-5) * g2_ref[...] + be2_ref[...]

    return pl.pallas_call(
        body,
        out_shape=jax.ShapeDtypeStruct((N, D), jnp.float32),
    )(aggp, xw2, dis, inv, h1, b2, Wg2, bg2, gamma2, beta2)


def kernel(x, edge_index, W1, b1, Wlin, blin, Wg1, bg1, gamma1, beta1,
           W2, b2, Wg2, bg2, gamma2, beta2):
    src, dst = edge_index[0], edge_index[1]
    pad = EP - src.shape[0]
    src2d = jnp.concatenate(
        [src, jnp.zeros((pad,), jnp.int32)]).reshape(EP // CH, CH)
    dst2d = jnp.concatenate(
        [dst, jnp.full((pad,), N, jnp.int32)]).reshape(EP // CH, CH)

    xw1 = _tc0(x, W1)
    degp = _deg_kernel(dst2d).reshape(NC, NPAD, 1)
    y1, dis, inv = _tc1(xw1, degp)
    aggp1 = _scatter_kernel(y1, src2d, dst2d)
    h1 = _tc2(aggp1, xw1, dis, inv, x,
              b1.reshape(1, D), Wg1, bg1.reshape(1, D),
              Wlin, blin.reshape(1, D),
              gamma1.reshape(1, D), beta1.reshape(1, D))
    xw2, y2 = _tc2b(h1, W2, dis)
    aggp2 = _scatter_kernel(y2, src2d, dst2d)
    return _tc3(aggp2, xw2, dis, inv, h1,
                b2.reshape(1, D), Wg2, bg2.reshape(1, D),
                gamma2.reshape(1, D), beta2.reshape(1, D))


# speedup vs baseline: 1.4782x; 1.4782x over previous
import functools

import jax
import jax.numpy as jnp
from jax import lax
from jax.experimental import pallas as pl
from jax.experimental.pallas import tpu as pltpu
from jax.experimental.pallas import tpu_sc as plsc

N = 10000
D = 128
NC = 2
NS = 16
NW = NC * NS
CH = 128
CPT = 80
EP = NW * CPT * CH
RPT = 640
NPAD = NS * RPT
ZR = 64
NBUF = 2
GC = 8
NG = CPT // GC
CPT0 = 144
CPT1 = 16
NG0 = CPT0 // GC
NG1 = CPT1 // GC

_MESH = dict(core_axis_name="c", subcore_axis_name="s")


def _deg_kernel(dst2d):
    @functools.partial(
        pl.kernel,
        out_type=jax.ShapeDtypeStruct((NC, NPAD), jnp.float32),
        mesh=plsc.VectorSubcoreMesh(**_MESH),
        scratch_types=[
            pltpu.VMEM((CPT, CH), jnp.int32),
            pltpu.VMEM((CH,), jnp.float32),
            pltpu.VMEM((RPT,), jnp.float32),
            pltpu.VMEM_SHARED((NPAD,), jnp.float32),
        ],
    )
    def k(dst_hbm, deg_hbm, dstv, ones_v, zb, acc):
        cid = lax.axis_index("c")
        sid = lax.axis_index("s")
        wid = sid * NC + cid
        for j in range(CH // 16):
            ones_v[pl.ds(16 * j, 16)] = jnp.ones((16,), jnp.float32)

        def zfill(i, carry):
            zb[pl.ds(i * 16, 16)] = jnp.zeros((16,), jnp.float32)
            return carry

        lax.fori_loop(0, RPT // 16, zfill, 0)
        pltpu.sync_copy(zb, acc.at[pl.ds(sid * RPT, RPT)])
        pltpu.sync_copy(dst_hbm.at[pl.ds(wid * CPT, CPT)], dstv)
        plsc.subcore_barrier()

        def body(j, carry):
            pltpu.sync_copy(ones_v, acc.at[dstv.at[j]], add=True)
            return carry

        lax.fori_loop(0, CPT, body, 0)
        plsc.subcore_barrier()
        pltpu.sync_copy(acc.at[pl.ds(sid * RPT, RPT)],
                        deg_hbm.at[cid].at[pl.ds(sid * RPT, RPT)])

    return k(dst2d)


def _scatter_kernel(y, src2d, dst2d):
    @functools.partial(
        pl.kernel,
        out_type=jax.ShapeDtypeStruct((NC, NPAD, D), jnp.float32),
        mesh=plsc.VectorSubcoreMesh(**_MESH),
        scratch_types=[
            pltpu.VMEM((2 * GC, CH), jnp.int32),
            pltpu.VMEM((2 * GC, CH), jnp.int32),
            pltpu.VMEM((NBUF, CH, D), jnp.float32),
            pltpu.VMEM((ZR, D), jnp.float32),
            pltpu.VMEM_SHARED((NPAD, D), jnp.float32),
            [pltpu.SemaphoreType.DMA] * (NBUF + 4),
        ],
    )
    def k(y_hbm, src_hbm, dst_hbm, agg_hbm, sidx, didx, rows, zb, acc, sems):
        cid = lax.axis_index("c")
        sid = lax.axis_index("s")
        gsem = sems[:NBUF]
        isem = sems[NBUF:]
        base = jnp.where(cid == 0, sid * CPT0, NS * CPT0 + sid * CPT1)
        ng = jnp.where(cid == 0, NG0, NG1)

        def zfill(i, carry):
            for j in range(D // 16):
                zb[i, pl.ds(16 * j, 16)] = jnp.zeros((16,), jnp.float32)
            return carry

        def idx_fetch(g, s):
            pltpu.async_copy(src_hbm.at[pl.ds(base + g * GC, GC)],
                             sidx.at[pl.ds(s * GC, GC)], isem[2 * s])
            pltpu.async_copy(dst_hbm.at[pl.ds(base + g * GC, GC)],
                             didx.at[pl.ds(s * GC, GC)], isem[2 * s + 1])

        def idx_wait(s):
            pltpu.make_async_copy(src_hbm.at[pl.ds(base, GC)],
                                  sidx.at[pl.ds(s * GC, GC)],
                                  isem[2 * s]).wait()
            pltpu.make_async_copy(dst_hbm.at[pl.ds(base, GC)],
                                  didx.at[pl.ds(s * GC, GC)],
                                  isem[2 * s + 1]).wait()

        def gather(j, b):
            return [
                pltpu.async_copy(
                    y_hbm.at[sidx.at[j].at[pl.ds(h * (CH // 2), CH // 2)]],
                    rows.at[b].at[pl.ds(h * (CH // 2), CH // 2)],
                    gsem[b])
                for h in range(2)
            ]

        def group(s, prefetch_g):
            idx_wait(s)
            cps = gather(s * GC, 0)
            for c in range(GC):
                nxt = None
                if c + 1 < GC:
                    nxt = gather(s * GC + c + 1, (c + 1) % NBUF)
                for cp in cps:
                    cp.wait()
                pltpu.sync_copy(rows.at[c % NBUF],
                                acc.at[didx.at[s * GC + c]], add=True)
                cps = nxt
            idx_fetch(prefetch_g, s)

        lax.fori_loop(0, ZR, zfill, 0)
        for c in range(RPT // ZR):
            pltpu.sync_copy(zb, acc.at[pl.ds(sid * RPT + c * ZR, ZR)])
        idx_fetch(0, 0)
        idx_fetch(1, 1)
        plsc.subcore_barrier()

        def body(g2, carry):
            g = g2 * 2
            group(0, jnp.minimum(g + 2, ng - 1))
            group(1, jnp.minimum(g + 3, ng - 1))
            return carry

        lax.fori_loop(0, ng // 2, body, 0)
        idx_wait(0)
        idx_wait(1)
        plsc.subcore_barrier()
        pltpu.sync_copy(acc.at[pl.ds(sid * RPT, RPT)],
                        agg_hbm.at[cid].at[pl.ds(sid * RPT, RPT)])

    return k(y, src2d, dst2d)


def _tc0(x, W1):
    def body(x_ref, w_ref, xw_ref):
        xw_ref[...] = jnp.dot(x_ref[...], w_ref[...],
                              preferred_element_type=jnp.float32)

    return pl.pallas_call(
        body,
        out_shape=jax.ShapeDtypeStruct((N, D), jnp.float32),
    )(x, W1)


def _tc1(xw, degp):
    def body(xw_ref, degp_ref, y_ref, dis_ref, inv_ref):
        deg = degp_ref[0, :N, :] + degp_ref[1, :N, :] + 1.0
        dis = lax.rsqrt(deg)
        inv = 1.0 / deg
        y_ref[...] = xw_ref[...] * dis
        dis_ref[...] = dis
        inv_ref[...] = inv

    return pl.pallas_call(
        body,
        out_shape=[
            jax.ShapeDtypeStruct((N, D), jnp.float32),
            jax.ShapeDtypeStruct((N, 1), jnp.float32),
            jax.ShapeDtypeStruct((N, 1), jnp.float32),
        ],
    )(xw, degp)


def _agg_from_partials(aggp_ref):
    return aggp_ref[0, :N, :] + aggp_ref[1, :N, :]


def _tc2(aggp, xw1, dis, inv, x, b1, Wg1, bg1, Wlin, blin, gamma1, beta1):
    def body(aggp_ref, xw1_ref, dis_ref, inv_ref, x_ref, b1_ref, Wg1_ref,
             bg1_ref, Wlin_ref, blin_ref, g1_ref, be1_ref, h_ref):
        agg = _agg_from_partials(aggp_ref)
        conv = agg * dis_ref[...] + xw1_ref[...] * inv_ref[...] + b1_ref[...]
        z = jnp.tanh(conv)
        g = jax.nn.sigmoid(
            jnp.dot(z, Wg1_ref[...], preferred_element_type=jnp.float32)
            + bg1_ref[...])
        h0 = jnp.dot(x_ref[...], Wlin_ref[...],
                     preferred_element_type=jnp.float32) + blin_ref[...]
        h = (1.0 - g) * h0 + g * z
        h = jnp.maximum(h, 0.0)
        m = jnp.mean(h, axis=0, keepdims=True)
        v = jnp.mean((h - m) * (h - m), axis=0, keepdims=True)
        h_ref[...] = (h - m) * lax.rsqrt(v + 1e-5) * g1_ref[...] + be1_ref[...]

    return pl.pallas_call(
        body,
        out_shape=jax.ShapeDtypeStruct((N, D), jnp.float32),
    )(aggp, xw1, dis, inv, x, b1, Wg1, bg1, Wlin, blin, gamma1, beta1)


def _tc2b(h, W2, dis):
    def body(h_ref, W2_ref, dis_ref, xw2_ref, y2_ref):
        xw2 = jnp.dot(h_ref[...], W2_ref[...],
                      preferred_element_type=jnp.float32)
        xw2_ref[...] = xw2
        y2_ref[...] = xw2 * dis_ref[...]

    return pl.pallas_call(
        body,
        out_shape=[
            jax.ShapeDtypeStruct((N, D), jnp.float32),
            jax.ShapeDtypeStruct((N, D), jnp.float32),
        ],
    )(h, W2, dis)


def _tc3(aggp, xw2, dis, inv, h1, b2, Wg2, bg2, gamma2, beta2):
    def body(aggp_ref, xw2_ref, dis_ref, inv_ref, h1_ref, b2_ref, Wg2_ref,
             bg2_ref, g2_ref, be2_ref, out_ref):
        agg = _agg_from_partials(aggp_ref)
        conv = agg * dis_ref[...] + xw2_ref[...] * inv_ref[...] + b2_ref[...]
        z = jnp.tanh(conv)
        g = jax.nn.sigmoid(
            jnp.dot(z, Wg2_ref[...], preferred_element_type=jnp.float32)
            + bg2_ref[...])
        h = (1.0 - g) * h1_ref[...] + g * z
        h = jnp.maximum(h, 0.0)
        m = jnp.mean(h, axis=0, keepdims=True)
        v = jnp.mean((h - m) * (h - m), axis=0, keepdims=True)
        out_ref[...] = (h - m) * lax.rsqrt(v + 1e-5) * g2_ref[...] + be2_ref[...]

    return pl.pallas_call(
        body,
        out_shape=jax.ShapeDtypeStruct((N, D), jnp.float32),
    )(aggp, xw2, dis, inv, h1, b2, Wg2, bg2, gamma2, beta2)


def kernel(x, edge_index, W1, b1, Wlin, blin, Wg1, bg1, gamma1, beta1,
           W2, b2, Wg2, bg2, gamma2, beta2):
    src, dst = edge_index[0], edge_index[1]
    pad = EP - src.shape[0]
    src2d = jnp.concatenate(
        [src, jnp.zeros((pad,), jnp.int32)]).reshape(EP // CH, CH)
    dst2d = jnp.concatenate(
        [dst, jnp.full((pad,), N, jnp.int32)]).reshape(EP // CH, CH)

    xw1 = _tc0(x, W1)
    degp = _deg_kernel(dst2d).reshape(NC, NPAD, 1)
    y1, dis, inv = _tc1(xw1, degp)
    aggp1 = _scatter_kernel(y1, src2d, dst2d)
    h1 = _tc2(aggp1, xw1, dis, inv, x,
              b1.reshape(1, D), Wg1, bg1.reshape(1, D),
              Wlin, blin.reshape(1, D),
              gamma1.reshape(1, D), beta1.reshape(1, D))
    xw2, y2 = _tc2b(h1, W2, dis)
    aggp2 = _scatter_kernel(y2, src2d, dst2d)
    return _tc3(aggp2, xw2, dis, inv, h1,
                b2.reshape(1, D), Wg2, bg2.reshape(1, D),
                gamma2.reshape(1, D), beta2.reshape(1, D))


# speedup vs baseline: 1.5012x; 1.0156x over previous
import functools

import jax
import jax.numpy as jnp
from jax import lax
from jax.experimental import pallas as pl
from jax.experimental.pallas import tpu as pltpu
from jax.experimental.pallas import tpu_sc as plsc

N = 10000
D = 128
NC = 2
NS = 16
NW = NC * NS
CH = 128
CPT = 80
EP = NW * CPT * CH
RPT = 640
NPAD = NS * RPT
ZR = 64
NBUF = 2
GC = 8
NG = CPT // GC
CPT0 = 152
CPT1 = 8
NG0 = CPT0 // GC
NG1 = CPT1 // GC

_MESH = dict(core_axis_name="c", subcore_axis_name="s")


def _deg_kernel(dst2d):
    @functools.partial(
        pl.kernel,
        out_type=jax.ShapeDtypeStruct((NC, NPAD), jnp.float32),
        mesh=plsc.VectorSubcoreMesh(**_MESH),
        scratch_types=[
            pltpu.VMEM((CPT, CH), jnp.int32),
            pltpu.VMEM((CH,), jnp.float32),
            pltpu.VMEM((RPT,), jnp.float32),
            pltpu.VMEM_SHARED((NPAD,), jnp.float32),
        ],
    )
    def k(dst_hbm, deg_hbm, dstv, ones_v, zb, acc):
        cid = lax.axis_index("c")
        sid = lax.axis_index("s")
        wid = sid * NC + cid
        for j in range(CH // 16):
            ones_v[pl.ds(16 * j, 16)] = jnp.ones((16,), jnp.float32)

        def zfill(i, carry):
            zb[pl.ds(i * 16, 16)] = jnp.zeros((16,), jnp.float32)
            return carry

        lax.fori_loop(0, RPT // 16, zfill, 0)
        pltpu.sync_copy(zb, acc.at[pl.ds(sid * RPT, RPT)])
        pltpu.sync_copy(dst_hbm.at[pl.ds(wid * CPT, CPT)], dstv)
        plsc.subcore_barrier()

        def body(j, carry):
            pltpu.sync_copy(ones_v, acc.at[dstv.at[j]], add=True)
            return carry

        lax.fori_loop(0, CPT, body, 0)
        plsc.subcore_barrier()
        pltpu.sync_copy(acc.at[pl.ds(sid * RPT, RPT)],
                        deg_hbm.at[cid].at[pl.ds(sid * RPT, RPT)])

    return k(dst2d)


def _scatter_kernel(y, src2d, dst2d):
    @functools.partial(
        pl.kernel,
        out_type=jax.ShapeDtypeStruct((NC, NPAD, D), jnp.float32),
        mesh=plsc.VectorSubcoreMesh(**_MESH),
        scratch_types=[
            pltpu.VMEM((2 * GC, CH), jnp.int32),
            pltpu.VMEM((2 * GC, CH), jnp.int32),
            pltpu.VMEM((NBUF, CH, D), jnp.float32),
            pltpu.VMEM((ZR, D), jnp.float32),
            pltpu.VMEM_SHARED((NPAD, D), jnp.float32),
            [pltpu.SemaphoreType.DMA] * (NBUF + 4),
        ],
    )
    def k(y_hbm, src_hbm, dst_hbm, agg_hbm, sidx, didx, rows, zb, acc, sems):
        cid = lax.axis_index("c")
        sid = lax.axis_index("s")
        gsem = sems[:NBUF]
        isem = sems[NBUF:]
        base = jnp.where(cid == 0, sid * CPT0, NS * CPT0 + sid * CPT1)
        ng = jnp.where(cid == 0, NG0, NG1)

        def zfill(i, carry):
            for j in range(D // 16):
                zb[i, pl.ds(16 * j, 16)] = jnp.zeros((16,), jnp.float32)
            return carry

        def idx_fetch(g, s):
            pltpu.async_copy(src_hbm.at[pl.ds(base + g * GC, GC)],
                             sidx.at[pl.ds(s * GC, GC)], isem[2 * s])
            pltpu.async_copy(dst_hbm.at[pl.ds(base + g * GC, GC)],
                             didx.at[pl.ds(s * GC, GC)], isem[2 * s + 1])

        def idx_wait(s):
            pltpu.make_async_copy(src_hbm.at[pl.ds(base, GC)],
                                  sidx.at[pl.ds(s * GC, GC)],
                                  isem[2 * s]).wait()
            pltpu.make_async_copy(dst_hbm.at[pl.ds(base, GC)],
                                  didx.at[pl.ds(s * GC, GC)],
                                  isem[2 * s + 1]).wait()

        def group(s, prefetch_g):
            idx_wait(s)
            cp = pltpu.async_copy(y_hbm.at[sidx.at[s * GC]], rows.at[0],
                                  gsem[0])
            for c in range(GC):
                nxt = None
                if c + 1 < GC:
                    nxt = pltpu.async_copy(y_hbm.at[sidx.at[s * GC + c + 1]],
                                           rows.at[(c + 1) % NBUF],
                                           gsem[(c + 1) % NBUF])
                cp.wait()
                pltpu.sync_copy(rows.at[c % NBUF],
                                acc.at[didx.at[s * GC + c]], add=True)
                cp = nxt
            idx_fetch(prefetch_g, s)

        lax.fori_loop(0, ZR, zfill, 0)
        for c in range(RPT // ZR):
            pltpu.sync_copy(zb, acc.at[pl.ds(sid * RPT + c * ZR, ZR)])
        idx_fetch(0, 0)
        idx_fetch(jnp.minimum(1, ng - 1), 1)
        plsc.subcore_barrier()

        def body(g2, carry):
            g = g2 * 2
            group(0, jnp.minimum(g + 2, ng - 1))
            group(1, jnp.minimum(g + 3, ng - 1))
            return carry

        lax.fori_loop(0, ng // 2, body, 0)

        @pl.when(ng % 2 == 1)
        def _():
            group(0, ng - 1)

        idx_wait(0)
        idx_wait(1)
        plsc.subcore_barrier()
        pltpu.sync_copy(acc.at[pl.ds(sid * RPT, RPT)],
                        agg_hbm.at[cid].at[pl.ds(sid * RPT, RPT)])

    return k(y, src2d, dst2d)


def _tc0(x, W1):
    def body(x_ref, w_ref, xw_ref):
        xw_ref[...] = jnp.dot(x_ref[...], w_ref[...],
                              preferred_element_type=jnp.float32)

    return pl.pallas_call(
        body,
        out_shape=jax.ShapeDtypeStruct((N, D), jnp.float32),
    )(x, W1)


def _tc1(xw, degp):
    def body(xw_ref, degp_ref, y_ref, dis_ref, inv_ref):
        deg = degp_ref[0, :N, :] + degp_ref[1, :N, :] + 1.0
        dis = lax.rsqrt(deg)
        inv = 1.0 / deg
        y_ref[...] = xw_ref[...] * dis
        dis_ref[...] = dis
        inv_ref[...] = inv

    return pl.pallas_call(
        body,
        out_shape=[
            jax.ShapeDtypeStruct((N, D), jnp.float32),
            jax.ShapeDtypeStruct((N, 1), jnp.float32),
            jax.ShapeDtypeStruct((N, 1), jnp.float32),
        ],
    )(xw, degp)


def _agg_from_partials(aggp_ref):
    return aggp_ref[0, :N, :] + aggp_ref[1, :N, :]


def _tc2(aggp, xw1, dis, inv, x, b1, Wg1, bg1, Wlin, blin, gamma1, beta1):
    def body(aggp_ref, xw1_ref, dis_ref, inv_ref, x_ref, b1_ref, Wg1_ref,
             bg1_ref, Wlin_ref, blin_ref, g1_ref, be1_ref, h_ref):
        agg = _agg_from_partials(aggp_ref)
        conv = agg * dis_ref[...] + xw1_ref[...] * inv_ref[...] + b1_ref[...]
        z = jnp.tanh(conv)
        g = jax.nn.sigmoid(
            jnp.dot(z, Wg1_ref[...], preferred_element_type=jnp.float32)
            + bg1_ref[...])
        h0 = jnp.dot(x_ref[...], Wlin_ref[...],
                     preferred_element_type=jnp.float32) + blin_ref[...]
        h = (1.0 - g) * h0 + g * z
        h = jnp.maximum(h, 0.0)
        m = jnp.mean(h, axis=0, keepdims=True)
        v = jnp.mean((h - m) * (h - m), axis=0, keepdims=True)
        h_ref[...] = (h - m) * lax.rsqrt(v + 1e-5) * g1_ref[...] + be1_ref[...]

    return pl.pallas_call(
        body,
        out_shape=jax.ShapeDtypeStruct((N, D), jnp.float32),
    )(aggp, xw1, dis, inv, x, b1, Wg1, bg1, Wlin, blin, gamma1, beta1)


def _tc2b(h, W2, dis):
    def body(h_ref, W2_ref, dis_ref, xw2_ref, y2_ref):
        xw2 = jnp.dot(h_ref[...], W2_ref[...],
                      preferred_element_type=jnp.float32)
        xw2_ref[...] = xw2
        y2_ref[...] = xw2 * dis_ref[...]

    return pl.pallas_call(
        body,
        out_shape=[
            jax.ShapeDtypeStruct((N, D), jnp.float32),
            jax.ShapeDtypeStruct((N, D), jnp.float32),
        ],
    )(h, W2, dis)


def _tc3(aggp, xw2, dis, inv, h1, b2, Wg2, bg2, gamma2, beta2):
    def body(aggp_ref, xw2_ref, dis_ref, inv_ref, h1_ref, b2_ref, Wg2_ref,
             bg2_ref, g2_ref, be2_ref, out_ref):
        agg = _agg_from_partials(aggp_ref)
        conv = agg * dis_ref[...] + xw2_ref[...] * inv_ref[...] + b2_ref[...]
        z = jnp.tanh(conv)
        g = jax.nn.sigmoid(
            jnp.dot(z, Wg2_ref[...], preferred_element_type=jnp.float32)
            + bg2_ref[...])
        h = (1.0 - g) * h1_ref[...] + g * z
        h = jnp.maximum(h, 0.0)
        m = jnp.mean(h, axis=0, keepdims=True)
        v = jnp.mean((h - m) * (h - m), axis=0, keepdims=True)
        out_ref[...] = (h - m) * lax.rsqrt(v + 1e-5) * g2_ref[...] + be2_ref[...]

    return pl.pallas_call(
        body,
        out_shape=jax.ShapeDtypeStruct((N, D), jnp.float32),
    )(aggp, xw2, dis, inv, h1, b2, Wg2, bg2, gamma2, beta2)


def kernel(x, edge_index, W1, b1, Wlin, blin, Wg1, bg1, gamma1, beta1,
           W2, b2, Wg2, bg2, gamma2, beta2):
    src, dst = edge_index[0], edge_index[1]
    pad = EP - src.shape[0]
    src2d = jnp.concatenate(
        [src, jnp.zeros((pad,), jnp.int32)]).reshape(EP // CH, CH)
    dst2d = jnp.concatenate(
        [dst, jnp.full((pad,), N, jnp.int32)]).reshape(EP // CH, CH)

    xw1 = _tc0(x, W1)
    degp = _deg_kernel(dst2d).reshape(NC, NPAD, 1)
    y1, dis, inv = _tc1(xw1, degp)
    aggp1 = _scatter_kernel(y1, src2d, dst2d)
    h1 = _tc2(aggp1, xw1, dis, inv, x,
              b1.reshape(1, D), Wg1, bg1.reshape(1, D),
              Wlin, blin.reshape(1, D),
              gamma1.reshape(1, D), beta1.reshape(1, D))
    xw2, y2 = _tc2b(h1, W2, dis)
    aggp2 = _scatter_kernel(y2, src2d, dst2d)
    return _tc3(aggp2, xw2, dis, inv, h1,
                b2.reshape(1, D), Wg2, bg2.reshape(1, D),
                gamma2.reshape(1, D), beta2.reshape(1, D))
